# baseline (speedup 1.0000x reference)
import functools

import jax
import jax.numpy as jnp
from jax import lax
from jax.experimental import pallas as pl
from jax.experimental.pallas import tpu as pltpu
from jax.experimental.pallas import tpu_sc as plsc

B = 4096
C = 26
V = 100000
H = 32
NU = 100000

NC = 2
NS = 16
NW = NC * NS
BPW = B // NW

UT = 782
BU = 23
UCH = UT // BU
PROWS = C * UT
PSIZE = PROWS * H * 128
DSIZE = PSIZE - (H - 1) * 128


def _tc_project_table(x_ref, w_ref, o_ref):
    y = lax.dot_general(w_ref[...], x_ref[0],
                        dimension_numbers=(((0,), (0,)), ((), ())),
                        preferred_element_type=jnp.float32)
    for u in range(BU):
        o_ref[u] = y[:, u * 128:(u + 1) * 128]


_mesh = plsc.VectorSubcoreMesh(core_axis_name="c", subcore_axis_name="s")


@functools.partial(
    pl.kernel,
    mesh=_mesh,
    out_type=jax.ShapeDtypeStruct((H, B), jnp.float32),
    scratch_types=[
        pltpu.VMEM((BPW,), jnp.int32),
        pltpu.VMEM((C, BPW), jnp.int32),
        pltpu.VMEM((2, H, BPW), jnp.float32),
        pltpu.VMEM((H, BPW), jnp.float32),
        pltpu.SemaphoreType.DMA,
        pltpu.SemaphoreType.DMA,
        pltpu.SemaphoreType.DMA,
        pltpu.SemaphoreType.DMA,
    ],
    compiler_params=pltpu.CompilerParams(use_tc_tiling_on_sc=False),
)
def _sc_gather_sum(uids_hbm, attr_t_hbm, p_hbm, out_hbm,
                   uids_v, attrs_v, colbuf, acc, sem1, semg0, semg1, semw):
    wid = lax.axis_index("s") * NC + lax.axis_index("c")
    base = wid * BPW
    pltpu.sync_copy(uids_hbm.at[pl.ds(base, BPW)], uids_v)

    cps = [pltpu.async_copy(attr_t_hbm.at[pl.ds(c * NU, NU)].at[uids_v],
                            attrs_v.at[c], sem1)
           for c in range(C)]
    for cp in cps:
        cp.wait()

    def addr_body(t, carry):
        c = t >> 3
        i = t & 7
        sl = pl.ds(i * 16, 16)
        v = attrs_v[c, sl]
        attrs_v[c, sl] = (c * (UT * H * 128) + (v >> 7) * (H * 128)
                          + (v & 127))
        return carry

    lax.fori_loop(0, C * (BPW // 16), addr_body, 0)

    semg = (semg0, semg1)

    def fire(c):
        buf = colbuf.at[c % 2]

        def body(h, carry):
            src = p_hbm.at[pl.ds(h * 128, DSIZE)]
            pltpu.async_copy(src.at[attrs_v.at[c]], buf.at[h], semg[c % 2])
            return carry

        lax.fori_loop(0, H, body, 0)

    def accumulate(c):
        buf = colbuf.at[c % 2]
        dummy = out_hbm.at[pl.ds(0, H), pl.ds(0, BPW)]
        pltpu.make_async_copy(dummy, buf, semg[c % 2]).wait()
        if c == 0:
            def cp_body(t, carry):
                h = t >> 3
                sl = pl.ds((t & 7) * 16, 16)
                acc[h, sl] = buf[h, sl]
                return carry

            lax.fori_loop(0, H * (BPW // 16), cp_body, 0)
        else:
            def add_body(t, carry):
                h = t >> 3
                sl = pl.ds((t & 7) * 16, 16)
                acc[h, sl] = acc[h, sl] + buf[h, sl]
                return carry

            lax.fori_loop(0, H * (BPW // 16), add_body, 0)

    fire(0)
    for c in range(1, C):
        fire(c)
        accumulate(c - 1)
    accumulate(C - 1)

    pltpu.async_copy(acc, out_hbm.at[pl.ds(0, H), pl.ds(base, BPW)], semw)
    dummy = out_hbm.at[pl.ds(0, H), pl.ds(0, BPW)]
    pltpu.make_async_copy(dummy, acc, semw).wait()


BB = 512


def _tc_finish(g_ref, ue_ref, w_ref, b_ref, eye_ref, o_ref):
    acc = jnp.dot(ue_ref[...], w_ref[...], preferred_element_type=jnp.float32)
    acc += lax.dot_general(g_ref[...], eye_ref[...],
                           dimension_numbers=(((0,), (0,)), ((), ())),
                           preferred_element_type=jnp.float32)
    o_ref[...] = acc + b_ref[...]


def kernel(uids, user_embedding, attr_table, embed_tables, W, b):
    attr_t = attr_table.T.reshape(-1)
    emb_t = embed_tables.transpose(0, 2, 1)

    pt = pl.pallas_call(
        _tc_project_table,
        grid=(C, UCH),
        in_specs=[pl.BlockSpec((1, H, BU * 128), lambda c, u: (c, 0, u)),
                  pl.BlockSpec((H, H), lambda c, u: (c + 1, 0))],
        out_specs=pl.BlockSpec((BU, H, 128), lambda c, u: (c * UCH + u, 0, 0)),
        out_shape=jax.ShapeDtypeStruct((PROWS, H, 128), jnp.float32),
    )(emb_t, W)
    p_flat = pt.reshape(-1)

    gsum = _sc_gather_sum(uids, attr_t, p_flat)

    out = pl.pallas_call(
        _tc_finish,
        grid=(B // BB,),
        in_specs=[
            pl.BlockSpec((H, BB), lambda i: (0, i)),
            pl.BlockSpec((BB, H), lambda i: (i, 0)),
            pl.BlockSpec((H, H), lambda i: (0, 0)),
            pl.BlockSpec((1, H), lambda i: (0, 0)),
            pl.BlockSpec((H, H), lambda i: (0, 0)),
        ],
        out_specs=pl.BlockSpec((BB, H), lambda i: (i, 0)),
        out_shape=jax.ShapeDtypeStruct((B, H), jnp.float32),
    )(gsum, user_embedding, W[0:H], b.reshape(1, H),
      jnp.eye(H, dtype=jnp.float32))
    return out


# speedup vs baseline: 1.4049x; 1.4049x over previous
import functools

import jax
import jax.numpy as jnp
from jax import lax
from jax.experimental import pallas as pl
from jax.experimental.pallas import tpu as pltpu
from jax.experimental.pallas import tpu_sc as plsc

B = 4096
C = 26
V = 100000
H = 32
NU = 100000

NC = 2
NS = 16
NW = NC * NS
BPW = B // NW

UT = 782
BU = 46
UCH = UT // BU
PROWS = C * UT
PSIZE = PROWS * H * 128
DSIZE = PSIZE - (H - 1) * 128


def _tc_project_table(x_ref, w_ref, o_ref):
    y = lax.dot_general(w_ref[...], x_ref[0],
                        dimension_numbers=(((0,), (0,)), ((), ())),
                        preferred_element_type=jnp.float32)
    for u in range(BU):
        o_ref[u] = y[:, u * 128:(u + 1) * 128]


_mesh = plsc.VectorSubcoreMesh(core_axis_name="c", subcore_axis_name="s")


@functools.partial(
    pl.kernel,
    mesh=_mesh,
    out_type=jax.ShapeDtypeStruct((H, B), jnp.float32),
    scratch_types=[
        pltpu.VMEM((BPW,), jnp.int32),
        pltpu.VMEM((C, BPW), jnp.int32),
        pltpu.VMEM((2, H, BPW), jnp.float32),
        pltpu.VMEM((H, BPW), jnp.float32),
        pltpu.SemaphoreType.DMA,
        pltpu.SemaphoreType.DMA,
        pltpu.SemaphoreType.DMA,
        pltpu.SemaphoreType.DMA,
    ],
    compiler_params=pltpu.CompilerParams(use_tc_tiling_on_sc=False),
)
def _sc_gather_sum(uids_hbm, attr_t_hbm, p_hbm, out_hbm,
                   uids_v, attrs_v, colbuf, acc, sem1, semg0, semg1, semw):
    wid = lax.axis_index("s") * NC + lax.axis_index("c")
    base = wid * BPW
    pltpu.sync_copy(uids_hbm.at[pl.ds(base, BPW)], uids_v)

    cps = [pltpu.async_copy(attr_t_hbm.at[pl.ds(c * NU, NU)].at[uids_v],
                            attrs_v.at[c], sem1)
           for c in range(C)]
    for cp in cps:
        cp.wait()

    def addr_body(t, carry):
        c = t >> 3
        i = t & 7
        sl = pl.ds(i * 16, 16)
        v = attrs_v[c, sl]
        attrs_v[c, sl] = (c * (UT * H * 128) + (v >> 7) * (H * 128)
                          + (v & 127))
        return carry

    lax.fori_loop(0, C * (BPW // 16), addr_body, 0)

    semg = (semg0, semg1)

    def fire(c):
        buf = colbuf.at[c % 2]

        def body(h, carry):
            src = p_hbm.at[pl.ds(h * 128, DSIZE)]
            pltpu.async_copy(src.at[attrs_v.at[c]], buf.at[h], semg[c % 2])
            return carry

        lax.fori_loop(0, H, body, 0)

    def accumulate(c):
        buf = colbuf.at[c % 2]
        dummy = out_hbm.at[pl.ds(0, H), pl.ds(0, BPW)]
        pltpu.make_async_copy(dummy, buf, semg[c % 2]).wait()
        if c == 0:
            def cp_body(t, carry):
                h = t >> 3
                sl = pl.ds((t & 7) * 16, 16)
                acc[h, sl] = buf[h, sl]
                return carry

            lax.fori_loop(0, H * (BPW // 16), cp_body, 0)
        else:
            def add_body(t, carry):
                h = t >> 3
                sl = pl.ds((t & 7) * 16, 16)
                acc[h, sl] = acc[h, sl] + buf[h, sl]
                return carry

            lax.fori_loop(0, H * (BPW // 16), add_body, 0)

    fire(0)
    for c in range(1, C):
        fire(c)
        accumulate(c - 1)
    accumulate(C - 1)

    pltpu.async_copy(acc, out_hbm.at[pl.ds(0, H), pl.ds(base, BPW)], semw)
    dummy = out_hbm.at[pl.ds(0, H), pl.ds(0, BPW)]
    pltpu.make_async_copy(dummy, acc, semw).wait()


BB = 512


def _tc_finish(g_ref, ue_ref, w_ref, b_ref, eye_ref, o_ref):
    acc = jnp.dot(ue_ref[...], w_ref[...], preferred_element_type=jnp.float32)
    acc += lax.dot_general(g_ref[...], eye_ref[...],
                           dimension_numbers=(((0,), (0,)), ((), ())),
                           preferred_element_type=jnp.float32)
    o_ref[...] = acc + b_ref[...]


def kernel(uids, user_embedding, attr_table, embed_tables, W, b):
    attr_t = attr_table.T.reshape(-1)
    emb_t = embed_tables.transpose(0, 2, 1)

    pt = pl.pallas_call(
        _tc_project_table,
        grid=(C, UCH),
        in_specs=[pl.BlockSpec((1, H, BU * 128), lambda c, u: (c, 0, u)),
                  pl.BlockSpec((H, H), lambda c, u: (c + 1, 0))],
        out_specs=pl.BlockSpec((BU, H, 128), lambda c, u: (c * UCH + u, 0, 0)),
        out_shape=jax.ShapeDtypeStruct((PROWS, H, 128), jnp.float32),
    )(emb_t, W)
    p_flat = pt.reshape(-1)

    gsum = _sc_gather_sum(uids, attr_t, p_flat)

    out = pl.pallas_call(
        _tc_finish,
        grid=(B // BB,),
        in_specs=[
            pl.BlockSpec((H, BB), lambda i: (0, i)),
            pl.BlockSpec((BB, H), lambda i: (i, 0)),
            pl.BlockSpec((H, H), lambda i: (0, 0)),
            pl.BlockSpec((1, H), lambda i: (0, 0)),
            pl.BlockSpec((H, H), lambda i: (0, 0)),
        ],
        out_specs=pl.BlockSpec((BB, H), lambda i: (i, 0)),
        out_shape=jax.ShapeDtypeStruct((B, H), jnp.float32),
    )(gsum, user_embedding, W[0:H], b.reshape(1, H),
      jnp.eye(H, dtype=jnp.float32))
    return out


# speedup vs baseline: 2.1618x; 1.5388x over previous
import functools

import jax
import jax.numpy as jnp
from jax import lax
from jax.experimental import pallas as pl
from jax.experimental.pallas import tpu as pltpu
from jax.experimental.pallas import tpu_sc as plsc

B = 4096
C = 26
V = 100000
H = 32
NU = 100000

NC = 2
NS = 16
NW = NC * NS
BPW = B // NW

UT = 782
BU = 391
UCH = UT // BU
PROWS = C * UT
PSIZE = PROWS * H * 128
DSIZE = PSIZE - (H - 1) * 128


def _tc_project_table(x_ref, w_ref, o_ref):
    y = lax.dot_general(w_ref[...], x_ref[0],
                        dimension_numbers=(((0,), (0,)), ((), ())),
                        preferred_element_type=jnp.float32)
    for u in range(BU):
        o_ref[u] = y[:, u * 128:(u + 1) * 128]


_mesh = plsc.VectorSubcoreMesh(core_axis_name="c", subcore_axis_name="s")


@functools.partial(
    pl.kernel,
    mesh=_mesh,
    out_type=jax.ShapeDtypeStruct((H, B), jnp.float32),
    scratch_types=[
        pltpu.VMEM((BPW,), jnp.int32),
        pltpu.VMEM((C, BPW), jnp.int32),
        pltpu.VMEM((2, H, BPW), jnp.float32),
        pltpu.VMEM((H, BPW), jnp.float32),
        pltpu.SemaphoreType.DMA,
        pltpu.SemaphoreType.DMA,
        pltpu.SemaphoreType.DMA,
        pltpu.SemaphoreType.DMA,
    ],
    compiler_params=pltpu.CompilerParams(use_tc_tiling_on_sc=False),
)
def _sc_gather_sum(uids_hbm, attr_t_hbm, p_hbm, out_hbm,
                   uids_v, attrs_v, colbuf, acc, sem1, semg0, semg1, semw):
    wid = lax.axis_index("s") * NC + lax.axis_index("c")
    base = wid * BPW
    pltpu.sync_copy(uids_hbm.at[pl.ds(base, BPW)], uids_v)

    cps = [pltpu.async_copy(attr_t_hbm.at[pl.ds(c * NU, NU)].at[uids_v],
                            attrs_v.at[c], sem1)
           for c in range(C)]
    for cp in cps:
        cp.wait()

    def addr_body(t, carry):
        c = t >> 3
        i = t & 7
        sl = pl.ds(i * 16, 16)
        v = attrs_v[c, sl]
        attrs_v[c, sl] = (c * (UT * H * 128) + (v >> 7) * (H * 128)
                          + (v & 127))
        return carry

    lax.fori_loop(0, C * (BPW // 16), addr_body, 0)

    semg = (semg0, semg1)

    def fire(c):
        buf = colbuf.at[c % 2]

        def body(h, carry):
            src = p_hbm.at[pl.ds(h * 128, DSIZE)]
            pltpu.async_copy(src.at[attrs_v.at[c]], buf.at[h], semg[c % 2])
            return carry

        lax.fori_loop(0, H, body, 0)

    def accumulate(c):
        buf = colbuf.at[c % 2]
        dummy = out_hbm.at[pl.ds(0, H), pl.ds(0, BPW)]
        pltpu.make_async_copy(dummy, buf, semg[c % 2]).wait()
        if c == 0:
            def cp_body(t, carry):
                h = t >> 3
                sl = pl.ds((t & 7) * 16, 16)
                acc[h, sl] = buf[h, sl]
                return carry

            lax.fori_loop(0, H * (BPW // 16), cp_body, 0)
        else:
            def add_body(t, carry):
                h = t >> 3
                sl = pl.ds((t & 7) * 16, 16)
                acc[h, sl] = acc[h, sl] + buf[h, sl]
                return carry

            lax.fori_loop(0, H * (BPW // 16), add_body, 0)

    fire(0)
    for c in range(1, C):
        fire(c)
        accumulate(c - 1)
    accumulate(C - 1)

    pltpu.async_copy(acc, out_hbm.at[pl.ds(0, H), pl.ds(base, BPW)], semw)
    dummy = out_hbm.at[pl.ds(0, H), pl.ds(0, BPW)]
    pltpu.make_async_copy(dummy, acc, semw).wait()


BB = 512


def _tc_finish(g_ref, ue_ref, w_ref, b_ref, eye_ref, o_ref):
    acc = jnp.dot(ue_ref[...], w_ref[...], preferred_element_type=jnp.float32)
    acc += lax.dot_general(g_ref[...], eye_ref[...],
                           dimension_numbers=(((0,), (0,)), ((), ())),
                           preferred_element_type=jnp.float32)
    o_ref[...] = acc + b_ref[...]


def kernel(uids, user_embedding, attr_table, embed_tables, W, b):
    attr_t = attr_table.T.reshape(-1)
    emb_t = embed_tables.transpose(0, 2, 1)

    pt = pl.pallas_call(
        _tc_project_table,
        grid=(C, UCH),
        in_specs=[pl.BlockSpec((1, H, BU * 128), lambda c, u: (c, 0, u)),
                  pl.BlockSpec((H, H), lambda c, u: (c + 1, 0))],
        out_specs=pl.BlockSpec((BU, H, 128), lambda c, u: (c * UCH + u, 0, 0)),
        out_shape=jax.ShapeDtypeStruct((PROWS, H, 128), jnp.float32),
    )(emb_t, W)
    p_flat = pt.reshape(-1)

    gsum = _sc_gather_sum(uids, attr_t, p_flat)

    out = pl.pallas_call(
        _tc_finish,
        grid=(B // BB,),
        in_specs=[
            pl.BlockSpec((H, BB), lambda i: (0, i)),
            pl.BlockSpec((BB, H), lambda i: (i, 0)),
            pl.BlockSpec((H, H), lambda i: (0, 0)),
            pl.BlockSpec((1, H), lambda i: (0, 0)),
            pl.BlockSpec((H, H), lambda i: (0, 0)),
        ],
        out_specs=pl.BlockSpec((BB, H), lambda i: (i, 0)),
        out_shape=jax.ShapeDtypeStruct((B, H), jnp.float32),
    )(gsum, user_embedding, W[0:H], b.reshape(1, H),
      jnp.eye(H, dtype=jnp.float32))
    return out


# speedup vs baseline: 2.1659x; 1.0019x over previous
import functools

import jax
import jax.numpy as jnp
from jax import lax
from jax.experimental import pallas as pl
from jax.experimental.pallas import tpu as pltpu
from jax.experimental.pallas import tpu_sc as plsc

B = 4096
C = 26
V = 100000
H = 32
NU = 100000

NC = 2
NS = 16
NW = NC * NS
BPW = B // NW

UT = 782
BU = 782
UCH = UT // BU
PROWS = C * UT
PSIZE = PROWS * H * 128
DSIZE = PSIZE - (H - 1) * 128


def _tc_project_table(x_ref, w_ref, o_ref):
    y = lax.dot_general(w_ref[...], x_ref[0],
                        dimension_numbers=(((0,), (0,)), ((), ())),
                        preferred_element_type=jnp.float32)
    for u in range(BU):
        o_ref[u] = y[:, u * 128:(u + 1) * 128]


_mesh = plsc.VectorSubcoreMesh(core_axis_name="c", subcore_axis_name="s")


@functools.partial(
    pl.kernel,
    mesh=_mesh,
    out_type=jax.ShapeDtypeStruct((H, B), jnp.float32),
    scratch_types=[
        pltpu.VMEM((BPW,), jnp.int32),
        pltpu.VMEM((C, BPW), jnp.int32),
        pltpu.VMEM((2, H, BPW), jnp.float32),
        pltpu.VMEM((H, BPW), jnp.float32),
        pltpu.SemaphoreType.DMA,
        pltpu.SemaphoreType.DMA,
        pltpu.SemaphoreType.DMA,
        pltpu.SemaphoreType.DMA,
    ],
    compiler_params=pltpu.CompilerParams(use_tc_tiling_on_sc=False),
)
def _sc_gather_sum(uids_hbm, attr_t_hbm, p_hbm, out_hbm,
                   uids_v, attrs_v, colbuf, acc, sem1, semg0, semg1, semw):
    wid = lax.axis_index("s") * NC + lax.axis_index("c")
    base = wid * BPW
    pltpu.sync_copy(uids_hbm.at[pl.ds(base, BPW)], uids_v)

    cps = [pltpu.async_copy(attr_t_hbm.at[pl.ds(c * NU, NU)].at[uids_v],
                            attrs_v.at[c], sem1)
           for c in range(C)]
    for cp in cps:
        cp.wait()

    def addr_body(t, carry):
        c = t >> 3
        i = t & 7
        sl = pl.ds(i * 16, 16)
        v = attrs_v[c, sl]
        attrs_v[c, sl] = (c * (UT * H * 128) + (v >> 7) * (H * 128)
                          + (v & 127))
        return carry

    lax.fori_loop(0, C * (BPW // 16), addr_body, 0)

    semg = (semg0, semg1)

    def fire(c):
        buf = colbuf.at[c % 2]

        def body(h, carry):
            src = p_hbm.at[pl.ds(h * 128, DSIZE)]
            pltpu.async_copy(src.at[attrs_v.at[c]], buf.at[h], semg[c % 2])
            return carry

        lax.fori_loop(0, H, body, 0)

    def accumulate(c):
        buf = colbuf.at[c % 2]
        dummy = out_hbm.at[pl.ds(0, H), pl.ds(0, BPW)]
        pltpu.make_async_copy(dummy, buf, semg[c % 2]).wait()
        if c == 0:
            def cp_body(t, carry):
                h = t >> 3
                sl = pl.ds((t & 7) * 16, 16)
                acc[h, sl] = buf[h, sl]
                return carry

            lax.fori_loop(0, H * (BPW // 16), cp_body, 0)
        else:
            def add_body(t, carry):
                h = t >> 3
                sl = pl.ds((t & 7) * 16, 16)
                acc[h, sl] = acc[h, sl] + buf[h, sl]
                return carry

            lax.fori_loop(0, H * (BPW // 16), add_body, 0)

    fire(0)
    for c in range(1, C):
        fire(c)
        accumulate(c - 1)
    accumulate(C - 1)

    pltpu.async_copy(acc, out_hbm.at[pl.ds(0, H), pl.ds(base, BPW)], semw)
    dummy = out_hbm.at[pl.ds(0, H), pl.ds(0, BPW)]
    pltpu.make_async_copy(dummy, acc, semw).wait()


BB = 512


def _tc_finish(g_ref, ue_ref, w_ref, b_ref, eye_ref, o_ref):
    acc = jnp.dot(ue_ref[...], w_ref[...], preferred_element_type=jnp.float32)
    acc += lax.dot_general(g_ref[...], eye_ref[...],
                           dimension_numbers=(((0,), (0,)), ((), ())),
                           preferred_element_type=jnp.float32)
    o_ref[...] = acc + b_ref[...]


def kernel(uids, user_embedding, attr_table, embed_tables, W, b):
    attr_t = attr_table.T.reshape(-1)
    emb_t = embed_tables.transpose(0, 2, 1)

    pt = pl.pallas_call(
        _tc_project_table,
        grid=(C, UCH),
        in_specs=[pl.BlockSpec((1, H, BU * 128), lambda c, u: (c, 0, u)),
                  pl.BlockSpec((H, H), lambda c, u: (c + 1, 0))],
        out_specs=pl.BlockSpec((BU, H, 128), lambda c, u: (c * UCH + u, 0, 0)),
        out_shape=jax.ShapeDtypeStruct((PROWS, H, 128), jnp.float32),
    )(emb_t, W)
    p_flat = pt.reshape(-1)

    gsum = _sc_gather_sum(uids, attr_t, p_flat)

    out = pl.pallas_call(
        _tc_finish,
        grid=(B // BB,),
        in_specs=[
            pl.BlockSpec((H, BB), lambda i: (0, i)),
            pl.BlockSpec((BB, H), lambda i: (i, 0)),
            pl.BlockSpec((H, H), lambda i: (0, 0)),
            pl.BlockSpec((1, H), lambda i: (0, 0)),
            pl.BlockSpec((H, H), lambda i: (0, 0)),
        ],
        out_specs=pl.BlockSpec((BB, H), lambda i: (i, 0)),
        out_shape=jax.ShapeDtypeStruct((B, H), jnp.float32),
    )(gsum, user_embedding, W[0:H], b.reshape(1, H),
      jnp.eye(H, dtype=jnp.float32))
    return out


# speedup vs baseline: 2.1821x; 1.0075x over previous
import functools

import jax
import jax.numpy as jnp
from jax import lax
from jax.experimental import pallas as pl
from jax.experimental.pallas import tpu as pltpu
from jax.experimental.pallas import tpu_sc as plsc

B = 4096
C = 26
V = 100000
H = 32
NU = 100000

NC = 2
NS = 16
NW = NC * NS
BPW = B // NW

UT = 782
BU = 782
UCH = UT // BU
PROWS = C * UT
PSIZE = PROWS * H * 128
DSIZE = PSIZE - (H - 1) * 128


def _tc_project_table(x_ref, w_ref, o_ref):
    y = lax.dot_general(w_ref[...], x_ref[0],
                        dimension_numbers=(((0,), (0,)), ((), ())),
                        preferred_element_type=jnp.float32)
    for u in range(BU):
        o_ref[u] = y[:, u * 128:(u + 1) * 128]


_mesh = plsc.VectorSubcoreMesh(core_axis_name="c", subcore_axis_name="s")


@functools.partial(
    pl.kernel,
    mesh=_mesh,
    out_type=jax.ShapeDtypeStruct((H, B), jnp.float32),
    scratch_types=[
        pltpu.VMEM((BPW,), jnp.int32),
        pltpu.VMEM((C, BPW), jnp.int32),
        pltpu.VMEM((4, H, BPW), jnp.float32),
        pltpu.VMEM((H, BPW), jnp.float32),
        pltpu.SemaphoreType.DMA,
        pltpu.SemaphoreType.DMA,
        pltpu.SemaphoreType.DMA,
        pltpu.SemaphoreType.DMA,
        pltpu.SemaphoreType.DMA,
        pltpu.SemaphoreType.DMA,
    ],
    compiler_params=pltpu.CompilerParams(use_tc_tiling_on_sc=False),
)
def _sc_gather_sum(uids_hbm, attr_t_hbm, p_hbm, out_hbm,
                   uids_v, attrs_v, colbuf, acc,
                   sem1, semg0, semg1, semg2, semg3, semw):
    wid = lax.axis_index("s") * NC + lax.axis_index("c")
    base = wid * BPW
    pltpu.sync_copy(uids_hbm.at[pl.ds(base, BPW)], uids_v)

    cps = [pltpu.async_copy(attr_t_hbm.at[pl.ds(c * NU, NU)].at[uids_v],
                            attrs_v.at[c], sem1)
           for c in range(C)]
    for cp in cps:
        cp.wait()

    def addr_body(t, carry):
        c = t >> 3
        i = t & 7
        sl = pl.ds(i * 16, 16)
        v = attrs_v[c, sl]
        attrs_v[c, sl] = (c * (UT * H * 128) + (v >> 7) * (H * 128)
                          + (v & 127))
        return carry

    lax.fori_loop(0, C * (BPW // 16), addr_body, 0)

    semg = (semg0, semg1, semg2, semg3)

    def fire(c):
        buf = colbuf.at[c % 4]

        def body(h, carry):
            src = p_hbm.at[pl.ds(h * 128, DSIZE)]
            pltpu.async_copy(src.at[attrs_v.at[c]], buf.at[h], semg[c % 4])
            return carry

        lax.fori_loop(0, H, body, 0)

    def accumulate(c):
        buf = colbuf.at[c % 4]
        dummy = out_hbm.at[pl.ds(0, H), pl.ds(0, BPW)]
        pltpu.make_async_copy(dummy, buf, semg[c % 4]).wait()
        if c == 0:
            def cp_body(t, carry):
                h = t >> 3
                sl = pl.ds((t & 7) * 16, 16)
                acc[h, sl] = buf[h, sl]
                return carry

            lax.fori_loop(0, H * (BPW // 16), cp_body, 0)
        else:
            def add_body(t, carry):
                h = t >> 3
                sl = pl.ds((t & 7) * 16, 16)
                acc[h, sl] = acc[h, sl] + buf[h, sl]
                return carry

            lax.fori_loop(0, H * (BPW // 16), add_body, 0)

    fire(0)
    fire(1)
    fire(2)
    for c in range(3, C):
        fire(c)
        accumulate(c - 3)
    accumulate(C - 3)
    accumulate(C - 2)
    accumulate(C - 1)

    pltpu.async_copy(acc, out_hbm.at[pl.ds(0, H), pl.ds(base, BPW)], semw)
    dummy = out_hbm.at[pl.ds(0, H), pl.ds(0, BPW)]
    pltpu.make_async_copy(dummy, acc, semw).wait()


BB = 512


def _tc_finish(g_ref, ue_ref, w_ref, b_ref, eye_ref, o_ref):
    acc = jnp.dot(ue_ref[...], w_ref[...], preferred_element_type=jnp.float32)
    acc += lax.dot_general(g_ref[...], eye_ref[...],
                           dimension_numbers=(((0,), (0,)), ((), ())),
                           preferred_element_type=jnp.float32)
    o_ref[...] = acc + b_ref[...]


def kernel(uids, user_embedding, attr_table, embed_tables, W, b):
    attr_t = attr_table.T.reshape(-1)
    emb_t = embed_tables.transpose(0, 2, 1)

    pt = pl.pallas_call(
        _tc_project_table,
        grid=(C, UCH),
        in_specs=[pl.BlockSpec((1, H, BU * 128), lambda c, u: (c, 0, u)),
                  pl.BlockSpec((H, H), lambda c, u: (c + 1, 0))],
        out_specs=pl.BlockSpec((BU, H, 128), lambda c, u: (c * UCH + u, 0, 0)),
        out_shape=jax.ShapeDtypeStruct((PROWS, H, 128), jnp.float32),
    )(emb_t, W)
    p_flat = pt.reshape(-1)

    gsum = _sc_gather_sum(uids, attr_t, p_flat)

    out = pl.pallas_call(
        _tc_finish,
        grid=(B // BB,),
        in_specs=[
            pl.BlockSpec((H, BB), lambda i: (0, i)),
            pl.BlockSpec((BB, H), lambda i: (i, 0)),
            pl.BlockSpec((H, H), lambda i: (0, 0)),
            pl.BlockSpec((1, H), lambda i: (0, 0)),
            pl.BlockSpec((H, H), lambda i: (0, 0)),
        ],
        out_specs=pl.BlockSpec((BB, H), lambda i: (i, 0)),
        out_shape=jax.ShapeDtypeStruct((B, H), jnp.float32),
    )(gsum, user_embedding, W[0:H], b.reshape(1, H),
      jnp.eye(H, dtype=jnp.float32))
    return out


# speedup vs baseline: 2.2618x; 1.0365x over previous
import functools

import jax
import jax.numpy as jnp
from jax import lax
from jax.experimental import pallas as pl
from jax.experimental.pallas import tpu as pltpu
from jax.experimental.pallas import tpu_sc as plsc

B = 4096
C = 26
V = 100000
H = 32
NU = 100000

NC = 2
NS = 16
NW = NC * NS
BPW = B // NW

UT = 782
BU = 782
UCH = UT // BU
PROWS = C * UT
PSIZE = PROWS * H * 128
DSIZE = PSIZE - (H - 1) * 128
CH2 = C // 2
PS_H = CH2 * UT * H * 128
DS_H = PS_H - (H - 1) * 128


def _tc_project_table(x_ref, w_ref, o_ref):
    y = lax.dot_general(w_ref[...], x_ref[0],
                        dimension_numbers=(((0,), (0,)), ((), ())),
                        preferred_element_type=jnp.float32)
    for u in range(BU):
        o_ref[u] = y[:, u * 128:(u + 1) * 128]


_mesh = plsc.VectorSubcoreMesh(core_axis_name="c", subcore_axis_name="s")


def _make_sc(cl):
  @functools.partial(
    pl.kernel,
    mesh=_mesh,
    out_type=jax.ShapeDtypeStruct((H, B), jnp.float32),
    scratch_types=[
        pltpu.VMEM((BPW,), jnp.int32),
        pltpu.VMEM((CH2, BPW), jnp.int32),
        pltpu.VMEM((4, H, BPW), jnp.float32),
        pltpu.VMEM((H, BPW), jnp.float32),
        pltpu.SemaphoreType.DMA,
        pltpu.SemaphoreType.DMA,
        pltpu.SemaphoreType.DMA,
        pltpu.SemaphoreType.DMA,
        pltpu.SemaphoreType.DMA,
        pltpu.SemaphoreType.DMA,
    ],
    compiler_params=pltpu.CompilerParams(use_tc_tiling_on_sc=False),
)
  def _sc_gather_sum(uids_hbm, attr_t_hbm, p_hbm, out_hbm,
                     uids_v, attrs_v, colbuf, acc,
                     sem1, semg0, semg1, semg2, semg3, semw):
    wid = lax.axis_index("s") * NC + lax.axis_index("c")
    base = wid * BPW
    pltpu.sync_copy(uids_hbm.at[pl.ds(base, BPW)], uids_v)

    cps = [pltpu.async_copy(attr_t_hbm.at[pl.ds((cl + c) * NU, NU)].at[uids_v],
                            attrs_v.at[c], sem1)
           for c in range(CH2)]
    for cp in cps:
        cp.wait()

    def addr_body(t, carry):
        c = t >> 3
        i = t & 7
        sl = pl.ds(i * 16, 16)
        v = attrs_v[c, sl]
        attrs_v[c, sl] = (c * (UT * H * 128) + (v >> 7) * (H * 128)
                          + (v & 127))
        return carry

    lax.fori_loop(0, CH2 * (BPW // 16), addr_body, 0)

    semg = (semg0, semg1, semg2, semg3)

    def fire(c):
        buf = colbuf.at[c % 4]

        def body(h, carry):
            src = p_hbm.at[pl.ds(h * 128, DS_H)]
            pltpu.async_copy(src.at[attrs_v.at[c]], buf.at[h], semg[c % 4])
            return carry

        lax.fori_loop(0, H, body, 0)

    def accumulate(c):
        buf = colbuf.at[c % 4]
        dummy = out_hbm.at[pl.ds(0, H), pl.ds(0, BPW)]
        pltpu.make_async_copy(dummy, buf, semg[c % 4]).wait()
        if c == 0:
            def cp_body(t, carry):
                h = t >> 3
                sl = pl.ds((t & 7) * 16, 16)
                acc[h, sl] = buf[h, sl]
                return carry

            lax.fori_loop(0, H * (BPW // 16), cp_body, 0)
        else:
            def add_body(t, carry):
                h = t >> 3
                sl = pl.ds((t & 7) * 16, 16)
                acc[h, sl] = acc[h, sl] + buf[h, sl]
                return carry

            lax.fori_loop(0, H * (BPW // 16), add_body, 0)

    fire(0)
    fire(1)
    fire(2)
    for c in range(3, CH2):
        fire(c)
        accumulate(c - 3)
    accumulate(CH2 - 3)
    accumulate(CH2 - 2)
    accumulate(CH2 - 1)

    pltpu.async_copy(acc, out_hbm.at[pl.ds(0, H), pl.ds(base, BPW)], semw)
    dummy = out_hbm.at[pl.ds(0, H), pl.ds(0, BPW)]
    pltpu.make_async_copy(dummy, acc, semw).wait()

  return _sc_gather_sum


_sc_half0 = _make_sc(0)
_sc_half1 = _make_sc(CH2)


BB = 512


def _tc_finish(g0_ref, g1_ref, ue_ref, w_ref, b_ref, eye_ref, o_ref):
    acc = jnp.dot(ue_ref[...], w_ref[...], preferred_element_type=jnp.float32)
    acc += lax.dot_general(g0_ref[...] + g1_ref[...], eye_ref[...],
                           dimension_numbers=(((0,), (0,)), ((), ())),
                           preferred_element_type=jnp.float32)
    o_ref[...] = acc + b_ref[...]


def kernel(uids, user_embedding, attr_table, embed_tables, W, b):
    attr_t = attr_table.T.reshape(-1)
    emb_t = embed_tables.transpose(0, 2, 1)

    def project(cl):
        return pl.pallas_call(
            _tc_project_table,
            grid=(CH2, UCH),
            in_specs=[
                pl.BlockSpec((1, H, BU * 128), lambda c, u: (c + cl, 0, u)),
                pl.BlockSpec((H, H), lambda c, u: (c + 1 + cl, 0))],
            out_specs=pl.BlockSpec((BU, H, 128),
                                   lambda c, u: (c * UCH + u, 0, 0)),
            out_shape=jax.ShapeDtypeStruct((CH2 * UT, H, 128), jnp.float32),
        )(emb_t, W)

    p0 = project(0).reshape(-1)
    p1 = project(CH2).reshape(-1)
    g0 = _sc_half0(uids, attr_t, p0)
    g1 = _sc_half1(uids, attr_t, p1)

    out = pl.pallas_call(
        _tc_finish,
        grid=(B // BB,),
        in_specs=[
            pl.BlockSpec((H, BB), lambda i: (0, i)),
            pl.BlockSpec((H, BB), lambda i: (0, i)),
            pl.BlockSpec((BB, H), lambda i: (i, 0)),
            pl.BlockSpec((H, H), lambda i: (0, 0)),
            pl.BlockSpec((1, H), lambda i: (0, 0)),
            pl.BlockSpec((H, H), lambda i: (0, 0)),
        ],
        out_specs=pl.BlockSpec((BB, H), lambda i: (i, 0)),
        out_shape=jax.ShapeDtypeStruct((B, H), jnp.float32),
    )(g0, g1, user_embedding, W[0:H], b.reshape(1, H),
      jnp.eye(H, dtype=jnp.float32))
    return out
